# baseline (device time: 53500 ns/iter reference)
import functools

import jax
import jax.numpy as jnp
from jax import lax
from jax.experimental import pallas as pl
from jax.experimental.pallas import tpu as pltpu

N_DEV = 4
N_TOK = 1024
D = 256
H = 512
E_GLOBAL = 16
E_LOCAL = 4
CAP = 204


def kernel(x, router_W, route_idx, expert_W):
    del router_W

    def body(x_ref, route_ref, ew_ref, out_ref,
             w_comm, c_comm, w_full, xm_all,
             w_send, w_recv, c_send, c_recv):
        my = lax.axis_index("i")
        left = lax.rem(my + N_DEV - 1, N_DEV)
        right = lax.rem(my + 1, N_DEV)

        barrier = pltpu.get_barrier_semaphore()
        for nbr in (left, right):
            pl.semaphore_signal(barrier, inc=1, device_id=(nbr,),
                                device_id_type=pl.DeviceIdType.MESH)
        pl.semaphore_wait(barrier, 2)

        w_own = jnp.reshape(ew_ref[...], (E_LOCAL * D, H)).astype(jnp.bfloat16)
        w_comm[0, :, :] = w_own
        w_full[pl.ds(my * E_LOCAL * D, E_LOCAL * D), :] = w_own

        e_ids = lax.broadcasted_iota(jnp.int32, (N_TOK, E_GLOBAL), 1)
        onehot = (route_ref[...] == e_ids).astype(jnp.float32)
        totals = jnp.sum(onehot, axis=0, keepdims=True)
        c_comm[0, 0:1, 0:E_GLOBAL] = totals

        prefix = jnp.zeros((1, E_GLOBAL), jnp.float32)
        for h in range(N_DEV - 1):
            w_rdma = pltpu.make_async_remote_copy(
                src_ref=w_comm.at[h], dst_ref=w_comm.at[h + 1],
                send_sem=w_send.at[h], recv_sem=w_recv.at[h],
                device_id=(right,), device_id_type=pl.DeviceIdType.MESH)
            c_rdma = pltpu.make_async_remote_copy(
                src_ref=c_comm.at[h], dst_ref=c_comm.at[h + 1],
                send_sem=c_send.at[h], recv_sem=c_recv.at[h],
                device_id=(right,), device_id_type=pl.DeviceIdType.MESH)
            w_rdma.start()
            c_rdma.start()
            w_rdma.wait()
            c_rdma.wait()
            origin = lax.rem(my - (h + 1) + N_DEV, N_DEV)
            w_full[pl.ds(origin * E_LOCAL * D, E_LOCAL * D), :] = (
                w_comm[h + 1, :, :])
            cnt = c_comm[h + 1, 0:1, 0:E_GLOBAL]
            prefix = prefix + jnp.where(origin < my, cnt, 0.0)

        tri = (lax.broadcasted_iota(jnp.int32, (N_TOK, N_TOK), 0)
               > lax.broadcasted_iota(jnp.int32, (N_TOK, N_TOK), 1)
               ).astype(jnp.float32)
        excl = jnp.dot(tri, onehot, preferred_element_type=jnp.float32)
        before = jnp.sum(onehot * (excl + prefix), axis=1, keepdims=True)
        keep = (before < CAP).astype(jnp.float32)
        mask = (onehot * keep).astype(jnp.bfloat16)

        x_bf = x_ref[...].astype(jnp.bfloat16)
        for e in range(E_GLOBAL):
            xm_all[:, e * D:(e + 1) * D] = x_bf * mask[:, e:e + 1]
        out_ref[...] = jnp.dot(xm_all[...], w_full[...],
                               preferred_element_type=jnp.float32)

        @functools.partial(pl.run_scoped,
                           exit_sem=pltpu.SemaphoreType.REGULAR)
        def _(exit_sem):
            for nbr in (left, right):
                pl.semaphore_signal(exit_sem, inc=1, device_id=(nbr,),
                                    device_id_type=pl.DeviceIdType.MESH)
            pl.semaphore_wait(exit_sem, 2)

    return pl.pallas_call(
        body,
        out_shape=jax.ShapeDtypeStruct((N_TOK, H), jnp.float32),
        in_specs=[
            pl.BlockSpec(memory_space=pltpu.VMEM),
            pl.BlockSpec(memory_space=pltpu.VMEM),
            pl.BlockSpec(memory_space=pltpu.VMEM),
        ],
        out_specs=pl.BlockSpec(memory_space=pltpu.VMEM),
        scratch_shapes=[
            pltpu.VMEM((N_DEV, E_LOCAL * D, H), jnp.bfloat16),
            pltpu.VMEM((N_DEV, 8, 128), jnp.float32),
            pltpu.VMEM((E_GLOBAL * D, H), jnp.bfloat16),
            pltpu.VMEM((N_TOK, E_GLOBAL * D), jnp.bfloat16),
            pltpu.SemaphoreType.DMA((N_DEV - 1,)),
            pltpu.SemaphoreType.DMA((N_DEV - 1,)),
            pltpu.SemaphoreType.DMA((N_DEV - 1,)),
            pltpu.SemaphoreType.DMA((N_DEV - 1,)),
        ],
        compiler_params=pltpu.CompilerParams(collective_id=0),
    )(x, route_idx, expert_W)


# device time: 32861 ns/iter; 1.6281x vs baseline; 1.6281x over previous
import functools

import jax
import jax.numpy as jnp
from jax import lax
from jax.experimental import pallas as pl
from jax.experimental.pallas import tpu as pltpu

N_DEV = 4
N_TOK = 1024
D = 256
H = 512
E_GLOBAL = 16
E_LOCAL = 4
CAP = 204
ROWS = E_LOCAL * D
HALF = ROWS // 2


def kernel(x, router_W, route_idx, expert_W):
    del router_W

    def body(x_ref, route_ref, ew_ref, out_ref,
             w_own, w_L, w_R, w_dgA, w_dgB, xm_all, c_slots,
             w_send, w_recv, c_send, c_recv):
        my = lax.axis_index("i")
        left = lax.rem(my + N_DEV - 1, N_DEV)
        right = lax.rem(my + 1, N_DEV)
        diag = lax.rem(my + 2, N_DEV)

        barrier = pltpu.get_barrier_semaphore()
        for nbr in (left, right):
            pl.semaphore_signal(barrier, inc=1, device_id=(nbr,),
                                device_id_type=pl.DeviceIdType.MESH)
        pl.semaphore_wait(barrier, 2)

        w_own[...] = jnp.reshape(ew_ref[...], (2, HALF, H)).astype(jnp.bfloat16)
        e_ids = lax.broadcasted_iota(jnp.int32, (N_TOK, E_GLOBAL), 1)
        onehot = (route_ref[...] == e_ids).astype(jnp.float32)
        c_slots[0, 0:1, 0:E_GLOBAL] = jnp.sum(onehot, axis=0, keepdims=True)

        c_to_r = pltpu.make_async_remote_copy(
            src_ref=c_slots.at[0], dst_ref=c_slots.at[1],
            send_sem=c_send.at[0], recv_sem=c_recv.at[0],
            device_id=(right,), device_id_type=pl.DeviceIdType.MESH)
        c_to_l = pltpu.make_async_remote_copy(
            src_ref=c_slots.at[0], dst_ref=c_slots.at[2],
            send_sem=c_send.at[1], recv_sem=c_recv.at[1],
            device_id=(left,), device_id_type=pl.DeviceIdType.MESH)
        w_to_r = pltpu.make_async_remote_copy(
            src_ref=w_own, dst_ref=w_L,
            send_sem=w_send.at[0], recv_sem=w_recv.at[0],
            device_id=(right,), device_id_type=pl.DeviceIdType.MESH)
        w_to_l = pltpu.make_async_remote_copy(
            src_ref=w_own, dst_ref=w_R,
            send_sem=w_send.at[1], recv_sem=w_recv.at[1],
            device_id=(left,), device_id_type=pl.DeviceIdType.MESH)
        c_to_r.start()
        c_to_l.start()
        w_to_r.start()
        w_to_l.start()

        rel_ids = lax.rem(4 * my + e_ids, E_GLOBAL)
        oh_rel = (route_ref[...] == rel_ids).astype(jnp.bfloat16)
        x_bf = x_ref[...].astype(jnp.bfloat16)
        for c in range(E_GLOBAL):
            xm_all[:, c * D:(c + 1) * D] = x_bf * oh_rel[:, c:c + 1]

        tri = (lax.broadcasted_iota(jnp.int32, (N_TOK, N_TOK), 0)
               > lax.broadcasted_iota(jnp.int32, (N_TOK, N_TOK), 1)
               ).astype(jnp.float32)
        excl = jnp.dot(tri, onehot, preferred_element_type=jnp.float32)

        out_ref[...] = jnp.dot(
            xm_all[:, 0:ROWS], jnp.reshape(w_own[...], (ROWS, H)),
            preferred_element_type=jnp.float32)

        c_to_r.wait_recv()
        c_fwd = pltpu.make_async_remote_copy(
            src_ref=c_slots.at[1], dst_ref=c_slots.at[3],
            send_sem=c_send.at[2], recv_sem=c_recv.at[2],
            device_id=(right,), device_id_type=pl.DeviceIdType.MESH)
        c_fwd.start()

        w_to_r.wait_recv()
        w_fwdA = pltpu.make_async_remote_copy(
            src_ref=w_L.at[0], dst_ref=w_dgA,
            send_sem=w_send.at[2], recv_sem=w_recv.at[2],
            device_id=(right,), device_id_type=pl.DeviceIdType.MESH)
        w_fwdA.start()
        out_ref[...] = out_ref[...] + jnp.dot(
            xm_all[:, 3 * ROWS:4 * ROWS], jnp.reshape(w_L[...], (ROWS, H)),
            preferred_element_type=jnp.float32)

        w_to_l.wait_recv()
        w_fwdB = pltpu.make_async_remote_copy(
            src_ref=w_R.at[1], dst_ref=w_dgB,
            send_sem=w_send.at[3], recv_sem=w_recv.at[3],
            device_id=(left,), device_id_type=pl.DeviceIdType.MESH)
        w_fwdB.start()
        out_ref[...] = out_ref[...] + jnp.dot(
            xm_all[:, ROWS:2 * ROWS], jnp.reshape(w_R[...], (ROWS, H)),
            preferred_element_type=jnp.float32)

        w_fwdA.wait_recv()
        out_ref[...] = out_ref[...] + jnp.dot(
            xm_all[:, 2 * ROWS:2 * ROWS + HALF], w_dgA[...],
            preferred_element_type=jnp.float32)
        w_fwdB.wait_recv()
        out_ref[...] = out_ref[...] + jnp.dot(
            xm_all[:, 2 * ROWS + HALF:3 * ROWS], w_dgB[...],
            preferred_element_type=jnp.float32)

        c_to_l.wait_recv()
        c_fwd.wait_recv()
        zero = jnp.zeros((1, E_GLOBAL), jnp.float32)
        prefix = (jnp.where(left < my, c_slots[1, 0:1, 0:E_GLOBAL], zero)
                  + jnp.where(right < my, c_slots[2, 0:1, 0:E_GLOBAL], zero)
                  + jnp.where(diag < my, c_slots[3, 0:1, 0:E_GLOBAL], zero))
        before = jnp.sum(onehot * (excl + prefix), axis=1, keepdims=True)
        keep = (before < CAP).astype(jnp.float32)
        out_ref[...] = out_ref[...] * keep

        for desc in (c_to_r, c_to_l, c_fwd, w_to_r, w_to_l, w_fwdA, w_fwdB):
            desc.wait_send()

        @functools.partial(pl.run_scoped,
                           exit_sem=pltpu.SemaphoreType.REGULAR)
        def _(exit_sem):
            for nbr in (left, right):
                pl.semaphore_signal(exit_sem, inc=1, device_id=(nbr,),
                                    device_id_type=pl.DeviceIdType.MESH)
            pl.semaphore_wait(exit_sem, 2)

    return pl.pallas_call(
        body,
        out_shape=jax.ShapeDtypeStruct((N_TOK, H), jnp.float32),
        in_specs=[
            pl.BlockSpec(memory_space=pltpu.VMEM),
            pl.BlockSpec(memory_space=pltpu.VMEM),
            pl.BlockSpec(memory_space=pltpu.VMEM),
        ],
        out_specs=pl.BlockSpec(memory_space=pltpu.VMEM),
        scratch_shapes=[
            pltpu.VMEM((2, HALF, H), jnp.bfloat16),
            pltpu.VMEM((2, HALF, H), jnp.bfloat16),
            pltpu.VMEM((2, HALF, H), jnp.bfloat16),
            pltpu.VMEM((HALF, H), jnp.bfloat16),
            pltpu.VMEM((HALF, H), jnp.bfloat16),
            pltpu.VMEM((N_TOK, E_GLOBAL * D), jnp.bfloat16),
            pltpu.VMEM((4, 8, 128), jnp.float32),
            pltpu.SemaphoreType.DMA((4,)),
            pltpu.SemaphoreType.DMA((4,)),
            pltpu.SemaphoreType.DMA((3,)),
            pltpu.SemaphoreType.DMA((3,)),
        ],
        compiler_params=pltpu.CompilerParams(collective_id=0),
    )(x, route_idx, expert_W)


# device time: 28079 ns/iter; 1.9053x vs baseline; 1.1703x over previous
import jax
import jax.numpy as jnp
from jax import lax
from jax.experimental import pallas as pl
from jax.experimental.pallas import tpu as pltpu

N_DEV = 4
N_TOK = 1024
D = 256
H = 512
E_GLOBAL = 16
E_LOCAL = 4
CAP = 204
ROWS = E_LOCAL * D
HALF = ROWS // 2


def kernel(x, router_W, route_idx, expert_W):
    del router_W
    x = x.astype(jnp.bfloat16)
    expert_W = expert_W.astype(jnp.bfloat16)

    def body(x_ref, route_ref, ew_ref, out_ref,
             w_L, w_R, w_dgA, w_dgB, xm_all, c_slots, acc,
             w_send, w_recv, c_send, c_recv):
        my = lax.axis_index("i")
        left = lax.rem(my + N_DEV - 1, N_DEV)
        right = lax.rem(my + 1, N_DEV)
        diag = lax.rem(my + 2, N_DEV)

        barrier = pltpu.get_barrier_semaphore()
        for nbr in (left, right):
            pl.semaphore_signal(barrier, inc=1, device_id=(nbr,),
                                device_id_type=pl.DeviceIdType.MESH)

        e_ids = lax.broadcasted_iota(jnp.int32, (N_TOK, E_GLOBAL), 1)
        onehot = (route_ref[...] == e_ids).astype(jnp.float32)
        c_slots[0, 0:1, 0:E_GLOBAL] = jnp.sum(onehot, axis=0, keepdims=True)

        pl.semaphore_wait(barrier, 2)

        c_to_r = pltpu.make_async_remote_copy(
            src_ref=c_slots.at[0], dst_ref=c_slots.at[1],
            send_sem=c_send.at[0], recv_sem=c_recv.at[0],
            device_id=(right,), device_id_type=pl.DeviceIdType.MESH)
        c_to_l = pltpu.make_async_remote_copy(
            src_ref=c_slots.at[0], dst_ref=c_slots.at[2],
            send_sem=c_send.at[1], recv_sem=c_recv.at[1],
            device_id=(left,), device_id_type=pl.DeviceIdType.MESH)
        c_to_d = pltpu.make_async_remote_copy(
            src_ref=c_slots.at[0], dst_ref=c_slots.at[3],
            send_sem=c_send.at[2], recv_sem=c_recv.at[2],
            device_id=(diag,), device_id_type=pl.DeviceIdType.MESH)
        w_to_r0 = pltpu.make_async_remote_copy(
            src_ref=ew_ref.at[pl.ds(0, 2)], dst_ref=w_L.at[0],
            send_sem=w_send.at[0], recv_sem=w_recv.at[0],
            device_id=(right,), device_id_type=pl.DeviceIdType.MESH)
        w_to_r1 = pltpu.make_async_remote_copy(
            src_ref=ew_ref.at[pl.ds(2, 2)], dst_ref=w_L.at[1],
            send_sem=w_send.at[1], recv_sem=w_recv.at[1],
            device_id=(right,), device_id_type=pl.DeviceIdType.MESH)
        w_to_l1 = pltpu.make_async_remote_copy(
            src_ref=ew_ref.at[pl.ds(2, 2)], dst_ref=w_R.at[1],
            send_sem=w_send.at[2], recv_sem=w_recv.at[2],
            device_id=(left,), device_id_type=pl.DeviceIdType.MESH)
        w_to_l0 = pltpu.make_async_remote_copy(
            src_ref=ew_ref.at[pl.ds(0, 2)], dst_ref=w_R.at[0],
            send_sem=w_send.at[3], recv_sem=w_recv.at[3],
            device_id=(left,), device_id_type=pl.DeviceIdType.MESH)
        c_to_r.start()
        c_to_l.start()
        c_to_d.start()
        w_to_r0.start()
        w_to_l1.start()
        w_to_r1.start()
        w_to_l0.start()

        rel_ids = lax.rem(4 * my + e_ids, E_GLOBAL)
        oh_rel = (route_ref[...] == rel_ids).astype(jnp.bfloat16)
        x_bf = x_ref[...]
        for c in range(E_GLOBAL):
            xm_all[:, c * D:(c + 1) * D] = x_bf * oh_rel[:, c:c + 1]

        tri = (lax.broadcasted_iota(jnp.int32, (N_TOK, N_TOK), 0)
               > lax.broadcasted_iota(jnp.int32, (N_TOK, N_TOK), 1)
               ).astype(jnp.float32)
        excl = jnp.dot(tri, onehot, preferred_element_type=jnp.float32)

        w_to_r0.wait_recv()
        w_fwdA = pltpu.make_async_remote_copy(
            src_ref=w_L.at[0], dst_ref=w_dgA,
            send_sem=w_send.at[4], recv_sem=w_recv.at[4],
            device_id=(right,), device_id_type=pl.DeviceIdType.MESH)
        w_fwdA.start()
        w_to_l1.wait_recv()
        w_fwdB = pltpu.make_async_remote_copy(
            src_ref=w_R.at[1], dst_ref=w_dgB,
            send_sem=w_send.at[5], recv_sem=w_recv.at[5],
            device_id=(left,), device_id_type=pl.DeviceIdType.MESH)
        w_fwdB.start()

        acc[...] = jnp.dot(
            xm_all[:, 0:ROWS], jnp.reshape(ew_ref[...], (ROWS, H)),
            preferred_element_type=jnp.float32)

        w_to_r1.wait_recv()
        acc[...] = acc[...] + jnp.dot(
            xm_all[:, 3 * ROWS:4 * ROWS], jnp.reshape(w_L[...], (ROWS, H)),
            preferred_element_type=jnp.float32)
        w_to_l0.wait_recv()
        acc[...] = acc[...] + jnp.dot(
            xm_all[:, ROWS:2 * ROWS], jnp.reshape(w_R[...], (ROWS, H)),
            preferred_element_type=jnp.float32)
        w_fwdA.wait_recv()
        acc[...] = acc[...] + jnp.dot(
            xm_all[:, 2 * ROWS:2 * ROWS + HALF],
            jnp.reshape(w_dgA[...], (HALF, H)),
            preferred_element_type=jnp.float32)
        w_fwdB.wait_recv()
        acc[...] = acc[...] + jnp.dot(
            xm_all[:, 2 * ROWS + HALF:3 * ROWS],
            jnp.reshape(w_dgB[...], (HALF, H)),
            preferred_element_type=jnp.float32)

        c_to_r.wait_recv()
        c_to_l.wait_recv()
        c_to_d.wait_recv()
        zero = jnp.zeros((1, E_GLOBAL), jnp.float32)
        prefix = (jnp.where(left < my, c_slots[1, 0:1, 0:E_GLOBAL], zero)
                  + jnp.where(right < my, c_slots[2, 0:1, 0:E_GLOBAL], zero)
                  + jnp.where(diag < my, c_slots[3, 0:1, 0:E_GLOBAL], zero))
        before = jnp.sum(onehot * (excl + prefix), axis=1, keepdims=True)
        keep = (before < CAP).astype(jnp.float32)
        out_ref[...] = (acc[...] * keep).astype(jnp.bfloat16)

        for desc in (c_to_r, c_to_l, c_to_d, w_to_r0, w_to_r1,
                     w_to_l0, w_to_l1, w_fwdA, w_fwdB):
            desc.wait_send()

    return pl.pallas_call(
        body,
        out_shape=jax.ShapeDtypeStruct((N_TOK, H), jnp.bfloat16),
        in_specs=[
            pl.BlockSpec(memory_space=pltpu.VMEM),
            pl.BlockSpec(memory_space=pltpu.VMEM),
            pl.BlockSpec(memory_space=pltpu.VMEM),
        ],
        out_specs=pl.BlockSpec(memory_space=pltpu.VMEM),
        scratch_shapes=[
            pltpu.VMEM((2, 2, D, H), jnp.bfloat16),
            pltpu.VMEM((2, 2, D, H), jnp.bfloat16),
            pltpu.VMEM((2, D, H), jnp.bfloat16),
            pltpu.VMEM((2, D, H), jnp.bfloat16),
            pltpu.VMEM((N_TOK, E_GLOBAL * D), jnp.bfloat16),
            pltpu.VMEM((4, 8, 128), jnp.float32),
            pltpu.VMEM((N_TOK, H), jnp.float32),
            pltpu.SemaphoreType.DMA((6,)),
            pltpu.SemaphoreType.DMA((6,)),
            pltpu.SemaphoreType.DMA((3,)),
            pltpu.SemaphoreType.DMA((3,)),
        ],
        compiler_params=pltpu.CompilerParams(collective_id=0),
    )(x, route_idx, expert_W)
